# baseline (speedup 1.0000x reference)
import functools

import jax
import jax.numpy as jnp
from jax import lax
from jax.experimental import pallas as pl
from jax.experimental.pallas import tpu as pltpu
from jax.experimental.pallas import tpu_sc as plsc

_D = 128
_NW = 32
_DPT = _D // _NW
_L = 16
_C = 2560
_UNROLL = 8


def _tc_pre(feat, W_pool, bp, WsT, bs):
    M, D = feat.shape
    def body(x_ref, wp_ref, bp_ref, ws_ref, bs_ref, ht_ref, s_ref):
        x = x_ref[...]
        hp = lax.dot_general(wp_ref[...], x, (((1,), (1,)), ((), ())),
                             preferred_element_type=jnp.float32)
        ht_ref[...] = jnp.maximum(hp + bp_ref[...], 0.0)
        sp = jnp.dot(x, ws_ref[...], preferred_element_type=jnp.float32)
        s_ref[...] = sp + bs_ref[...]
    return pl.pallas_call(
        body,
        out_shape=[
            jax.ShapeDtypeStruct((D, M), jnp.float32),
            jax.ShapeDtypeStruct((M, D), jnp.float32),
        ],
    )(feat, W_pool, bp.reshape(D, 1), WsT, bs.reshape(1, D))


def _tc_post(selfpart, neigh_t, W_neigh, bn):
    M, D = selfpart.shape
    def body(s_ref, n_ref, w_ref, b_ref, o_ref):
        nm = lax.dot_general(n_ref[...], w_ref[...], (((0,), (1,)), ((), ())),
                             preferred_element_type=jnp.float32)
        o_ref[...] = s_ref[...] + nm + b_ref[...]
    return pl.pallas_call(
        body,
        out_shape=jax.ShapeDtypeStruct((M, D), jnp.float32),
    )(selfpart, neigh_t, W_neigh, bn.reshape(1, D))


def _sc_agg(h_t, src, dst, w):
    D, N = h_t.shape
    E = src.shape[0]
    nchunk = E // _C
    assert nchunk * _C == E, "edge count must divide the scan chunk"
    assert N % _L == 0 and D == _D

    mesh = plsc.VectorSubcoreMesh(core_axis_name="c", subcore_axis_name="s")

    ngroups = _C // (_L * _UNROLL)

    @functools.partial(
        pl.kernel,
        out_type=jax.ShapeDtypeStruct((D, N), jnp.float32),
        mesh=mesh,
        scratch_types=(
            [pltpu.VMEM((N,), jnp.float32)] * _DPT +
            [pltpu.VMEM((N,), jnp.float32)] * _DPT +
            [
                pltpu.VMEM((_C,), jnp.int32),
                pltpu.VMEM((_C,), jnp.int32),
                pltpu.VMEM((_C,), jnp.float32),
                pltpu.VMEM((ngroups * _L,), jnp.int32),
                pltpu.SemaphoreType.DMA,
            ]
        ),
        compiler_params=pltpu.CompilerParams(needs_layout_passes=False),
    )
    def sc_kernel(ht_hbm, src_hbm, dst_hbm, w_hbm, out_hbm,
                  hb0, hb1, hb2, hb3, ac0, ac1, ac2, ac3,
                  sbuf, dbuf, wbuf, fbuf, sem):
        hbs = [hb0, hb1, hb2, hb3]
        acs = [ac0, ac1, ac2, ac3]
        wid = lax.axis_index("s") * 2 + lax.axis_index("c")
        d0 = wid * _DPT
        neg = jnp.float32(-jnp.inf)
        iota = lax.iota(jnp.int32, _L)

        for d4 in range(_DPT):
            pltpu.sync_copy(ht_hbm.at[d0 + d4], hbs[d4])

        def init_body(j, _):
            for d4 in range(_DPT):
                acs[d4][pl.ds(j * _L, _L)] = jnp.full((_L,), neg, jnp.float32)
            return 0
        lax.fori_loop(0, N // _L, init_body, 0)

        start = (wid * nchunk) // _NW
        prev = ((iota - 1) * (iota > 0)).reshape(_L, 1)
        gdn = lax.GatherDimensionNumbers(offset_dims=(),
                                         collapsed_slice_dims=(0,),
                                         start_index_map=(0,))

        def chunk_body(j, _):
            off = lax.rem(start + j, nchunk) * _C
            pltpu.sync_copy(src_hbm.at[pl.ds(off, _C)], sbuf)
            pltpu.sync_copy(dst_hbm.at[pl.ds(off, _C)], dbuf)
            pltpu.sync_copy(w_hbm.at[pl.ds(off, _C)], wbuf)

            def group_body(g, _):
                conf = None
                for u in range(_UNROLL):
                    b = (g * _UNROLL + u) * _L
                    srcv = sbuf[pl.ds(b, _L)]
                    dstv = dbuf[pl.ds(b, _L)]
                    wv = wbuf[pl.ds(b, _L)]
                    sk, _sv = plsc.sort_key_val(dstv, dstv)
                    rot = lax.gather(
                        sk, prev, gdn, (1,),
                        mode=lax.GatherScatterMode.PROMISE_IN_BOUNDS)
                    dup = ((sk == rot) & (iota >= 1)).astype(jnp.int32)
                    conf = dup if conf is None else (conf | dup)
                    for d4 in range(_DPT):
                        msg = plsc.load_gather(hbs[d4], [srcv]) * wv
                        a = plsc.load_gather(acs[d4], [dstv])
                        plsc.store_scatter(acs[d4], [dstv],
                                           jnp.maximum(a, msg))
                fbuf[pl.ds(g * _L, _L)] = conf
                return 0
            lax.fori_loop(0, ngroups, group_body, 0)

            def repair_vreg(v, _):
                b = v * _L
                srcv = sbuf[pl.ds(b, _L)]
                dstv = dbuf[pl.ds(b, _L)]
                wv = wbuf[pl.ds(b, _L)]
                for l in range(_L):
                    ssp = jnp.full((_L,), srcv[l], jnp.int32)
                    dsp = jnp.full((_L,), dstv[l], jnp.int32)
                    wsp = wv[l]
                    for d4 in range(_DPT):
                        msg = plsc.load_gather(hbs[d4], [ssp]) * wsp
                        a = plsc.load_gather(acs[d4], [dsp])
                        plsc.store_scatter(acs[d4], [dsp],
                                           jnp.maximum(a, msg))
                return 0

            def scan_body(g, _):
                conf = fbuf[pl.ds(g * _L, _L)]

                @pl.when(jnp.any(conf != 0))
                def _():
                    lax.fori_loop(g * _UNROLL, (g + 1) * _UNROLL,
                                  repair_vreg, 0)
                return 0
            lax.fori_loop(0, ngroups, scan_body, 0)
            return 0
        lax.fori_loop(0, nchunk, chunk_body, 0)

        def fix_body(j, _):
            for d4 in range(_DPT):
                a = acs[d4][pl.ds(j * _L, _L)]
                acs[d4][pl.ds(j * _L, _L)] = jnp.where(a == neg, 0.0, a)
            return 0
        lax.fori_loop(0, N // _L, fix_body, 0)
        for d4 in range(_DPT):
            pltpu.sync_copy(acs[d4], out_hbm.at[d0 + d4])

    return sc_kernel(h_t, src, dst, w)


def kernel(feat, edge_index, weight, W_pool, b_pool, W_self, b_self,
           W_neigh, b_neigh):
    h_t, selfpart = _tc_pre(feat, W_pool, b_pool, W_self.T, b_self)
    neigh_t = _sc_agg(h_t, edge_index[0], edge_index[1], weight[:, 0])
    return _tc_post(selfpart, neigh_t, W_neigh, b_neigh)


# speedup vs baseline: 1.3141x; 1.3141x over previous
import functools

import jax
import jax.numpy as jnp
from jax import lax
from jax.experimental import pallas as pl
from jax.experimental.pallas import tpu as pltpu
from jax.experimental.pallas import tpu_sc as plsc

_D = 128
_NW = 32
_DPT = _D // _NW
_L = 16
_C = 2560
_UNROLL = 8


def _tc_pre(feat, W_pool, bp, WsT, bs):
    M, D = feat.shape
    def body(x_ref, wp_ref, bp_ref, ws_ref, bs_ref, ht_ref, s_ref):
        x = x_ref[...]
        hp = lax.dot_general(wp_ref[...], x, (((1,), (1,)), ((), ())),
                             preferred_element_type=jnp.float32)
        ht_ref[...] = jnp.maximum(hp + bp_ref[...], 0.0)
        sp = jnp.dot(x, ws_ref[...], preferred_element_type=jnp.float32)
        s_ref[...] = sp + bs_ref[...]
    return pl.pallas_call(
        body,
        out_shape=[
            jax.ShapeDtypeStruct((D, M), jnp.float32),
            jax.ShapeDtypeStruct((M, D), jnp.float32),
        ],
    )(feat, W_pool, bp.reshape(D, 1), WsT, bs.reshape(1, D))


def _tc_post(selfpart, neigh_t, W_neigh, bn):
    M, D = selfpart.shape
    def body(s_ref, n_ref, w_ref, b_ref, o_ref):
        nm = lax.dot_general(n_ref[...], w_ref[...], (((0,), (1,)), ((), ())),
                             preferred_element_type=jnp.float32)
        o_ref[...] = s_ref[...] + nm + b_ref[...]
    return pl.pallas_call(
        body,
        out_shape=jax.ShapeDtypeStruct((M, D), jnp.float32),
    )(selfpart, neigh_t, W_neigh, bn.reshape(1, D))


def _sc_agg(h_t, src, dst, w):
    D, N = h_t.shape
    E = src.shape[0]
    nchunk = E // _C
    assert nchunk * _C == E, "edge count must divide the scan chunk"
    assert N % _L == 0 and D == _D

    mesh = plsc.VectorSubcoreMesh(core_axis_name="c", subcore_axis_name="s")

    ngroups = _C // (_L * _UNROLL)

    @functools.partial(
        pl.kernel,
        out_type=jax.ShapeDtypeStruct((D, N), jnp.float32),
        mesh=mesh,
        scratch_types=(
            [pltpu.VMEM((N,), jnp.float32)] * _DPT +
            [pltpu.VMEM((N,), jnp.float32)] * _DPT +
            [
                pltpu.VMEM((_C,), jnp.int32),
                pltpu.VMEM((_C,), jnp.int32),
                pltpu.VMEM((_C,), jnp.float32),
                pltpu.VMEM((ngroups * _L,), jnp.int32),
                pltpu.SemaphoreType.DMA,
            ]
        ),
        compiler_params=pltpu.CompilerParams(needs_layout_passes=False),
    )
    def sc_kernel(ht_hbm, src_hbm, dst_hbm, w_hbm, out_hbm,
                  hb0, hb1, hb2, hb3, ac0, ac1, ac2, ac3,
                  sbuf, dbuf, wbuf, fbuf, sem):
        hbs = [hb0, hb1, hb2, hb3]
        acs = [ac0, ac1, ac2, ac3]
        wid = lax.axis_index("s") * 2 + lax.axis_index("c")
        d0 = wid * _DPT
        neg = jnp.float32(-jnp.inf)
        iota = lax.iota(jnp.int32, _L)

        for d4 in range(_DPT):
            pltpu.sync_copy(ht_hbm.at[d0 + d4], hbs[d4])

        def init_body(j, _):
            for d4 in range(_DPT):
                acs[d4][pl.ds(j * _L, _L)] = jnp.full((_L,), neg, jnp.float32)
            return 0
        lax.fori_loop(0, N // _L, init_body, 0)

        start = (wid * nchunk) // _NW

        def chunk_body(j, _):
            off = lax.rem(start + j, nchunk) * _C
            pltpu.sync_copy(src_hbm.at[pl.ds(off, _C)], sbuf)
            pltpu.sync_copy(dst_hbm.at[pl.ds(off, _C)], dbuf)
            pltpu.sync_copy(w_hbm.at[pl.ds(off, _C)], wbuf)

            def group_body(g, _):
                conf = None
                for u in range(_UNROLL):
                    b = (g * _UNROLL + u) * _L
                    srcv = sbuf[pl.ds(b, _L)]
                    dstv = dbuf[pl.ds(b, _L)]
                    wv = wbuf[pl.ds(b, _L)]
                    msgs = []
                    for d4 in range(_DPT):
                        msg = plsc.load_gather(hbs[d4], [srcv]) * wv
                        a = plsc.load_gather(acs[d4], [dstv])
                        plsc.store_scatter(acs[d4], [dstv],
                                           jnp.maximum(a, msg))
                        msgs.append(msg)
                    bad = None
                    for d4 in range(_DPT):
                        a2 = plsc.load_gather(acs[d4], [dstv])
                        f = (a2 < msgs[d4]).astype(jnp.int32)
                        bad = f if bad is None else (bad | f)
                    conf = bad if conf is None else (conf | bad)
                fbuf[pl.ds(g * _L, _L)] = conf
                return 0
            lax.fori_loop(0, ngroups, group_body, 0)

            def repair_vreg(v, _):
                b = v * _L
                srcv = sbuf[pl.ds(b, _L)]
                dstv = dbuf[pl.ds(b, _L)]
                wv = wbuf[pl.ds(b, _L)]
                for l in range(_L):
                    ssp = jnp.full((_L,), srcv[l], jnp.int32)
                    dsp = jnp.full((_L,), dstv[l], jnp.int32)
                    wsp = wv[l]
                    for d4 in range(_DPT):
                        msg = plsc.load_gather(hbs[d4], [ssp]) * wsp
                        a = plsc.load_gather(acs[d4], [dsp])
                        plsc.store_scatter(acs[d4], [dsp],
                                           jnp.maximum(a, msg))
                return 0

            def scan_body(g, _):
                conf = fbuf[pl.ds(g * _L, _L)]

                @pl.when(jnp.any(conf != 0))
                def _():
                    lax.fori_loop(g * _UNROLL, (g + 1) * _UNROLL,
                                  repair_vreg, 0)
                return 0
            lax.fori_loop(0, ngroups, scan_body, 0)
            return 0
        lax.fori_loop(0, nchunk, chunk_body, 0)

        def fix_body(j, _):
            for d4 in range(_DPT):
                a = acs[d4][pl.ds(j * _L, _L)]
                acs[d4][pl.ds(j * _L, _L)] = jnp.where(a == neg, 0.0, a)
            return 0
        lax.fori_loop(0, N // _L, fix_body, 0)
        for d4 in range(_DPT):
            pltpu.sync_copy(acs[d4], out_hbm.at[d0 + d4])

    return sc_kernel(h_t, src, dst, w)


def kernel(feat, edge_index, weight, W_pool, b_pool, W_self, b_self,
           W_neigh, b_neigh):
    h_t, selfpart = _tc_pre(feat, W_pool, b_pool, W_self.T, b_self)
    neigh_t = _sc_agg(h_t, edge_index[0], edge_index[1], weight[:, 0])
    return _tc_post(selfpart, neigh_t, W_neigh, b_neigh)


# speedup vs baseline: 1.4511x; 1.1043x over previous
import functools

import jax
import jax.numpy as jnp
from jax import lax
from jax.experimental import pallas as pl
from jax.experimental.pallas import tpu as pltpu
from jax.experimental.pallas import tpu_sc as plsc

_D = 128
_NW = 32
_DPT = _D // _NW
_L = 16
_C = 2560
_UNROLL = 8


def _tc_pre(feat, W_pool, bp, WsT, bs):
    M, D = feat.shape
    def body(x_ref, wp_ref, bp_ref, ws_ref, bs_ref, ht_ref, s_ref):
        x = x_ref[...]
        hp = lax.dot_general(wp_ref[...], x, (((1,), (1,)), ((), ())),
                             preferred_element_type=jnp.float32)
        ht_ref[...] = jnp.maximum(hp + bp_ref[...], 0.0)
        sp = jnp.dot(x, ws_ref[...], preferred_element_type=jnp.float32)
        s_ref[...] = sp + bs_ref[...]
    return pl.pallas_call(
        body,
        out_shape=[
            jax.ShapeDtypeStruct((D, M), jnp.float32),
            jax.ShapeDtypeStruct((M, D), jnp.float32),
        ],
    )(feat, W_pool, bp.reshape(D, 1), WsT, bs.reshape(1, D))


def _tc_post(selfpart, neigh_t, W_neigh, bn):
    M, D = selfpart.shape
    def body(s_ref, n_ref, w_ref, b_ref, o_ref):
        nm = lax.dot_general(n_ref[...], w_ref[...], (((0,), (1,)), ((), ())),
                             preferred_element_type=jnp.float32)
        o_ref[...] = s_ref[...] + nm + b_ref[...]
    return pl.pallas_call(
        body,
        out_shape=jax.ShapeDtypeStruct((M, D), jnp.float32),
    )(selfpart, neigh_t, W_neigh, bn.reshape(1, D))


def _sc_agg(h_t, src, dst, w):
    D, N = h_t.shape
    E = src.shape[0]
    nchunk = E // _C
    assert nchunk * _C == E, "edge count must divide the scan chunk"
    assert N % _L == 0 and D == _D

    mesh = plsc.VectorSubcoreMesh(core_axis_name="c", subcore_axis_name="s")

    ngroups = _C // (_L * _UNROLL)

    @functools.partial(
        pl.kernel,
        out_type=jax.ShapeDtypeStruct((D, N), jnp.float32),
        mesh=mesh,
        scratch_types=(
            [pltpu.VMEM((N,), jnp.float32)] * _DPT +
            [pltpu.VMEM((N,), jnp.float32)] * _DPT +
            [
                pltpu.VMEM((_C,), jnp.int32),
                pltpu.VMEM((_C,), jnp.int32),
                pltpu.VMEM((_C,), jnp.float32),
                pltpu.VMEM((ngroups * _L,), jnp.int32),
                pltpu.VMEM_SHARED((E,), jnp.int32),
                pltpu.VMEM_SHARED((E,), jnp.int32),
                pltpu.SemaphoreType.DMA,
            ]
        ),
        compiler_params=pltpu.CompilerParams(needs_layout_passes=False),
    )
    def sc_kernel(ht_hbm, src_hbm, dst_hbm, w_hbm, out_hbm,
                  hb0, hb1, hb2, hb3, ac0, ac1, ac2, ac3,
                  sbuf, dbuf, wbuf, fbuf, s_sh, d_sh, sem):
        hbs = [hb0, hb1, hb2, hb3]
        acs = [ac0, ac1, ac2, ac3]
        wid = lax.axis_index("s") * 2 + lax.axis_index("c")
        d0 = wid * _DPT
        neg = jnp.float32(-jnp.inf)
        iota = lax.iota(jnp.int32, _L)

        @pl.when(lax.axis_index("s") == 0)
        def _():
            pltpu.sync_copy(src_hbm, s_sh)
            pltpu.sync_copy(dst_hbm, d_sh)

        for d4 in range(_DPT):
            pltpu.sync_copy(ht_hbm.at[d0 + d4], hbs[d4])
        plsc.subcore_barrier()

        def init_body(j, _):
            for d4 in range(_DPT):
                acs[d4][pl.ds(j * _L, _L)] = jnp.full((_L,), neg, jnp.float32)
            return 0
        lax.fori_loop(0, N // _L, init_body, 0)

        start = (wid * nchunk) // _NW

        def chunk_body(j, _):
            off = lax.rem(start + j, nchunk) * _C
            pltpu.sync_copy(s_sh.at[pl.ds(off, _C)], sbuf)
            pltpu.sync_copy(d_sh.at[pl.ds(off, _C)], dbuf)
            pltpu.sync_copy(w_hbm.at[pl.ds(off, _C)], wbuf)

            def group_body(g, _):
                conf = None
                for u in range(_UNROLL):
                    b = (g * _UNROLL + u) * _L
                    srcv = sbuf[pl.ds(b, _L)]
                    dstv = dbuf[pl.ds(b, _L)]
                    wv = wbuf[pl.ds(b, _L)]
                    msgs = []
                    for d4 in range(_DPT):
                        msg = plsc.load_gather(hbs[d4], [srcv]) * wv
                        a = plsc.load_gather(acs[d4], [dstv])
                        plsc.store_scatter(acs[d4], [dstv],
                                           jnp.maximum(a, msg))
                        msgs.append(msg)
                    bad = None
                    for d4 in range(_DPT):
                        a2 = plsc.load_gather(acs[d4], [dstv])
                        f = (a2 < msgs[d4]).astype(jnp.int32)
                        bad = f if bad is None else (bad | f)
                    conf = bad if conf is None else (conf | bad)
                fbuf[pl.ds(g * _L, _L)] = conf
                return 0
            lax.fori_loop(0, ngroups, group_body, 0)

            def repair_vreg(v, _):
                b = v * _L
                srcv = sbuf[pl.ds(b, _L)]
                dstv = dbuf[pl.ds(b, _L)]
                wv = wbuf[pl.ds(b, _L)]
                for l in range(_L):
                    ssp = jnp.full((_L,), srcv[l], jnp.int32)
                    dsp = jnp.full((_L,), dstv[l], jnp.int32)
                    wsp = wv[l]
                    for d4 in range(_DPT):
                        msg = plsc.load_gather(hbs[d4], [ssp]) * wsp
                        a = plsc.load_gather(acs[d4], [dsp])
                        plsc.store_scatter(acs[d4], [dsp],
                                           jnp.maximum(a, msg))
                return 0

            def scan_body(g, _):
                conf = fbuf[pl.ds(g * _L, _L)]

                @pl.when(jnp.any(conf != 0))
                def _():
                    lax.fori_loop(g * _UNROLL, (g + 1) * _UNROLL,
                                  repair_vreg, 0)
                return 0
            lax.fori_loop(0, ngroups, scan_body, 0)
            return 0
        lax.fori_loop(0, nchunk, chunk_body, 0)

        def fix_body(j, _):
            for d4 in range(_DPT):
                a = acs[d4][pl.ds(j * _L, _L)]
                acs[d4][pl.ds(j * _L, _L)] = jnp.where(a == neg, 0.0, a)
            return 0
        lax.fori_loop(0, N // _L, fix_body, 0)
        for d4 in range(_DPT):
            pltpu.sync_copy(acs[d4], out_hbm.at[d0 + d4])

    return sc_kernel(h_t, src, dst, w)


def kernel(feat, edge_index, weight, W_pool, b_pool, W_self, b_self,
           W_neigh, b_neigh):
    h_t, selfpart = _tc_pre(feat, W_pool, b_pool, W_self.T, b_self)
    neigh_t = _sc_agg(h_t, edge_index[0], edge_index[1], weight[:, 0])
    return _tc_post(selfpart, neigh_t, W_neigh, b_neigh)
